# baseline (speedup 1.0000x reference)
import jax
import jax.numpy as jnp
from jax import lax
from jax.experimental import pallas as pl
from jax.experimental.pallas import tpu as pltpu

M = 16384
N = 16384
BM = 128
NBUF = 4
NSTEPS = M // BM


def _mv_body(a_hbm, x_ref, o_ref, b0, b1, b2, b3, s0, s1, s2, s3):
    bufs = (b0, b1, b2, b3)
    sems = (s0, s1, s2, s3)

    def start(k, b):
        pltpu.async_copy(a_hbm.at[pl.ds(k * BM, BM), :], bufs[b], sems[b])

    def wait(k, b):
        pltpu.make_async_copy(a_hbm.at[pl.ds(k * BM, BM), :], bufs[b],
                              sems[b]).wait()

    for b in range(NBUF):
        start(b, b)

    xv = x_ref[...]

    def gbody(g, carry):
        base = g * NBUF
        for b in range(NBUF):
            k = base + b
            wait(k, b)
            o_ref[pl.ds(k * BM, BM)] = jnp.sum(bufs[b][...] * xv, axis=1)

            @pl.when(k + NBUF < NSTEPS)
            def _():
                start(k + NBUF, b)

        return carry

    lax.fori_loop(0, NSTEPS // NBUF, gbody, 0)


def kernel(A, x):
    return pl.pallas_call(
        _mv_body,
        in_specs=[
            pl.BlockSpec(memory_space=pltpu.HBM),
            pl.BlockSpec((1, N), lambda: (0, 0)),
        ],
        out_specs=pl.BlockSpec((M,), lambda: (0,)),
        out_shape=jax.ShapeDtypeStruct((M,), jnp.float32),
        scratch_shapes=(
            [pltpu.VMEM((BM, N), jnp.float32) for _ in range(NBUF)]
            + [pltpu.SemaphoreType.DMA for _ in range(NBUF)]
        ),
    )(A, x.reshape(1, N))
